# baseline (speedup 1.0000x reference)
import functools

import jax
import jax.numpy as jnp
from jax import lax
from jax.experimental import pallas as pl
from jax.experimental.pallas import tpu as pltpu
from jax.experimental.pallas import tpu_sc as plsc

B = 8
H = 512
N = 128
NLAYERS = 16
PADL = NLAYERS + 1
NPROPS = 4096


def _sigmoid(v):
    return 1.0 / (1.0 + jnp.exp(-v))


def _chain_body(x_ref, w1_ref, b1_ref, w2_ref, b2_ref, pos_ref, diag_ref):
    xb = x_ref[0]
    f32 = jnp.float32
    ii = lax.broadcasted_iota(jnp.int32, (H, H), 0)
    jj = lax.broadcasted_iota(jnp.int32, (H, H), 1)
    eye_h = (ii == jj).astype(f32)
    ii2 = lax.broadcasted_iota(jnp.int32, (N, N), 0)
    jj2 = lax.broadcasted_iota(jnp.int32, (N, N), 1)
    eye_n = (ii2 == jj2).astype(f32)

    dn = (((0,), (0,)), ((), ()))
    y = lax.dot_general(xb, eye_h, dn, preferred_element_type=f32)
    pos_ref[0, 0] = y.astype(jnp.bfloat16)
    diag_ref[0, 0] = xb

    w1 = w1_ref[...]
    w2 = w2_ref[...]
    b1 = b1_ref[...]
    b2 = b2_ref[...]
    dmm = (((1,), (1,)), ((), ()))
    row_iota = lax.broadcasted_iota(jnp.int32, (N, 1), 0)
    bf16 = jnp.bfloat16
    for d in range(1, NLAYERS):
        g1 = _sigmoid(lax.dot_general(y, w1, dmm, preferred_element_type=f32) + b1)
        g2 = _sigmoid(lax.dot_general(y, w2, dmm, preferred_element_type=f32) + b2)
        y_s = jnp.concatenate([y[1:], y[N - 1:]], axis=0)
        g2_s = jnp.concatenate([g2[1:], g2[N - 1:]], axis=0)
        y = g2_s * y + g1 * y_s
        ym = y * (row_iota < (N - d)).astype(f32)
        pos_ref[0, d] = ym.astype(bf16)
        diag_ref[0, d] = lax.dot_general(ym, eye_n, dn, preferred_element_type=f32)
    pos_ref[0, NLAYERS] = jnp.zeros((N, H), dtype=bf16)


def _chain(x, w1, b1, w2, b2):
    return pl.pallas_call(
        _chain_body,
        grid=(B,),
        in_specs=[
            pl.BlockSpec((1, H, N), lambda b: (b, 0, 0)),
            pl.BlockSpec((H, H), lambda b: (0, 0)),
            pl.BlockSpec((1, H), lambda b: (0, 0)),
            pl.BlockSpec((H, H), lambda b: (0, 0)),
            pl.BlockSpec((1, H), lambda b: (0, 0)),
        ],
        out_specs=[
            pl.BlockSpec((1, PADL, N, H), lambda b: (b, 0, 0, 0)),
            pl.BlockSpec((1, NLAYERS, H, N), lambda b: (b, 0, 0, 0)),
        ],
        out_shape=[
            jax.ShapeDtypeStruct((B, PADL, N, H), jnp.bfloat16),
            jax.ShapeDtypeStruct((B, NLAYERS, H, N), jnp.float32),
        ],
    )(x, w1, b1, w2, b2)


HTILE = 64


def _assemble_body(diag_ref, map_ref, mask_ref):
    f32 = jnp.float32
    band = diag_ref[0]
    v = []
    for d in range(NLAYERS):
        t = band[d]
        if d:
            t = jnp.concatenate([t[:, N - d:], t[:, : N - d]], axis=1)
        v.append(t)
    cmod = lax.broadcasted_iota(jnp.int32, (1, N), 1) % NLAYERS
    wlist = []
    for u in range(NLAYERS):
        acc = jnp.where(cmod == 0, v[(-u) % NLAYERS], 0.0)
        for k in range(1, NLAYERS):
            acc = acc + jnp.where(cmod == k, v[(k - u) % NLAYERS], 0.0)
        wlist.append(acc)
    wstack = jnp.stack(wlist, axis=1)
    x = jnp.broadcast_to(
        wstack[:, None], (HTILE, N // NLAYERS, NLAYERS, N)
    ).reshape(HTILE, N, N)
    row2 = lax.broadcasted_iota(jnp.int32, (N, N), 0)
    col2 = lax.broadcasted_iota(jnp.int32, (N, N), 1)
    diff = col2 - row2
    maskf = ((diff >= 0) & (diff < NLAYERS)).astype(f32)
    map_ref[0] = x * maskf[None]
    mask_ref[0, 0] = maskf


def _assemble(band_diag):
    return pl.pallas_call(
        _assemble_body,
        grid=(B, H // HTILE),
        in_specs=[
            pl.BlockSpec((1, NLAYERS, HTILE, N), lambda b, h: (b, 0, h, 0)),
        ],
        out_specs=[
            pl.BlockSpec((1, HTILE, N, N), lambda b, h: (b, h, 0, 0)),
            pl.BlockSpec((1, 1, N, N), lambda b, h: (b, 0, 0, 0)),
        ],
        out_shape=[
            jax.ShapeDtypeStruct((B, H, N, N), jnp.float32),
            jax.ShapeDtypeStruct((B, 1, N, N), jnp.float32),
        ],
    )(band_diag)


_ROWS_PER_B = PADL * N
_ZERO_ROW = NLAYERS * N
_NWORKERS = 32
_PER_W = (B * NPROPS) // _NWORKERS
_CHUNK = 64
_NCH = _PER_W // _CHUNK
_NBUF = 3


def _gather_body(table_hbm, s_hbm, e_hbm, out_hbm,
                 s_v, e_v, idx_v, rows0, rows1, rows2,
                 gs0, gs1, gs2, ws0, ws1, ws2):
    cid = lax.axis_index("c")
    sid = lax.axis_index("s")
    wid = sid * 2 + cid
    b = wid // (NPROPS // _PER_W)
    p0 = (wid % (NPROPS // _PER_W)) * _PER_W
    pltpu.sync_copy(s_hbm.at[pl.ds(p0, _PER_W)], s_v)
    pltpu.sync_copy(e_hbm.at[pl.ds(p0, _PER_W)], e_v)
    base = b * _ROWS_PER_B
    for i in range(_PER_W // 16):
        sv = s_v[pl.ds(i * 16, 16)]
        ev = e_v[pl.ds(i * 16, 16)]
        e1 = ev - 1
        e1 = jnp.where(e1 < 0, e1 + N, e1)
        d = e1 - sv
        valid = (d >= 0) & (d < NLAYERS)
        rows = jnp.where(valid, base + d * N + sv, base + _ZERO_ROW)
        idx_v[i // (_CHUNK // 16), pl.ds((i % (_CHUNK // 16)) * 16, 16)] = rows
    out0 = wid * _PER_W
    bufs = (rows0, rows1, rows2)
    gsems = (gs0, gs1, gs2)
    wsems = (ws0, ws1, ws2)

    def start_gather(t):
        m = t % _NBUF
        return pltpu.async_copy(
            table_hbm.at[idx_v.at[t]],
            bufs[m], gsems[m])

    h_g = [None] * _NCH
    h_w = [None] * _NCH
    for t in range(min(_NBUF - 1, _NCH)):
        h_g[t] = start_gather(t)
    for j in range(_NCH):
        h_g[j].wait()
        h_w[j] = pltpu.async_copy(
            bufs[j % _NBUF], out_hbm.at[pl.ds(out0 + j * _CHUNK, _CHUNK)],
            wsems[j % _NBUF])
        t = j + _NBUF - 1
        if t < _NCH:
            if j >= 1:
                h_w[j - 1].wait()
            h_g[t] = start_gather(t)
    for j in range(max(0, _NCH - _NBUF), _NCH):
        h_w[j].wait()


@functools.cache
def _make_gather():
    return pl.kernel(
        _gather_body,
        mesh=plsc.VectorSubcoreMesh(core_axis_name="c", subcore_axis_name="s"),
        out_type=jax.ShapeDtypeStruct((B * NPROPS, H // 2), jnp.int32),
        scratch_types=[
            pltpu.VMEM((_PER_W,), jnp.int32),
            pltpu.VMEM((_PER_W,), jnp.int32),
            pltpu.VMEM((_NCH, _CHUNK), jnp.int32),
            pltpu.VMEM((_CHUNK, H // 2), jnp.int32),
            pltpu.VMEM((_CHUNK, H // 2), jnp.int32),
            pltpu.VMEM((_CHUNK, H // 2), jnp.int32),
            pltpu.SemaphoreType.DMA,
            pltpu.SemaphoreType.DMA,
            pltpu.SemaphoreType.DMA,
            pltpu.SemaphoreType.DMA,
            pltpu.SemaphoreType.DMA,
            pltpu.SemaphoreType.DMA,
        ],
    )


def _cvt_body(in_ref, out_ref):
    out_ref[...] = in_ref[...].astype(jnp.float32)


def _cvt(x_bf):
    rows = B * NPROPS
    blk = rows // 32
    return pl.pallas_call(
        _cvt_body,
        grid=(32,),
        in_specs=[pl.BlockSpec((blk, H), lambda i: (i, 0))],
        out_specs=pl.BlockSpec((blk, H), lambda i: (i, 0)),
        out_shape=jax.ShapeDtypeStruct((rows, H), jnp.float32),
    )(x_bf)


def kernel(x, props, textual_input, textual_mask, W1, b1, W2, b2):
    del textual_input, textual_mask
    band_pos, band_diag = _chain(
        x, W1, b1.reshape(1, H), W2, b2.reshape(1, H)
    )
    ori_map_h, ori_map_mask = _assemble(band_diag)
    table = lax.bitcast_convert_type(
        band_pos.reshape(B * _ROWS_PER_B, H // 2, 2), jnp.int32)
    s_arr = props[:, 0].astype(jnp.int32)
    e_arr = props[:, 1].astype(jnp.int32)
    props_i = _make_gather()(table, s_arr, e_arr)
    props_bf = lax.bitcast_convert_type(props_i, jnp.bfloat16).reshape(
        B * NPROPS, H)
    props_h = _cvt(props_bf).reshape(B, NPROPS, H)
    return props_h, ori_map_h, ori_map_mask


# speedup vs baseline: 2.0375x; 2.0375x over previous
import functools

import jax
import jax.numpy as jnp
from jax import lax
from jax.experimental import pallas as pl
from jax.experimental.pallas import tpu as pltpu
from jax.experimental.pallas import tpu_sc as plsc

B = 8
H = 512
N = 128
NLAYERS = 16
PADL = NLAYERS + 1
NPROPS = 4096


def _sigmoid(v):
    return 1.0 / (1.0 + jnp.exp(-v))


def _chain_body(x_ref, w1_ref, b1_ref, w2_ref, b2_ref, pos_ref, diag_ref):
    xb = x_ref[0]
    f32 = jnp.float32
    ii = lax.broadcasted_iota(jnp.int32, (H, H), 0)
    jj = lax.broadcasted_iota(jnp.int32, (H, H), 1)
    eye_h = (ii == jj).astype(f32)
    ii2 = lax.broadcasted_iota(jnp.int32, (N, N), 0)
    jj2 = lax.broadcasted_iota(jnp.int32, (N, N), 1)
    eye_n = (ii2 == jj2).astype(f32)

    dn = (((0,), (0,)), ((), ()))
    y = lax.dot_general(xb, eye_h, dn, preferred_element_type=f32)
    pos_ref[0, 0] = y
    diag_ref[0, 0] = xb

    w1 = w1_ref[...]
    w2 = w2_ref[...]
    b1 = b1_ref[...]
    b2 = b2_ref[...]
    dmm = (((1,), (1,)), ((), ()))
    row_iota = lax.broadcasted_iota(jnp.int32, (N, 1), 0)
    for d in range(1, NLAYERS):
        g1 = _sigmoid(lax.dot_general(y, w1, dmm, preferred_element_type=f32) + b1)
        g2 = _sigmoid(lax.dot_general(y, w2, dmm, preferred_element_type=f32) + b2)
        y_s = jnp.concatenate([y[1:], y[N - 1:]], axis=0)
        g2_s = jnp.concatenate([g2[1:], g2[N - 1:]], axis=0)
        y = g2_s * y + g1 * y_s
        ym = y * (row_iota < (N - d)).astype(f32)
        pos_ref[0, d] = ym
        diag_ref[0, d] = lax.dot_general(ym, eye_n, dn, preferred_element_type=f32)
    pos_ref[0, NLAYERS] = jnp.zeros((N, H), dtype=f32)


def _chain(x, w1, b1, w2, b2):
    return pl.pallas_call(
        _chain_body,
        grid=(B,),
        in_specs=[
            pl.BlockSpec((1, H, N), lambda b: (b, 0, 0)),
            pl.BlockSpec((H, H), lambda b: (0, 0)),
            pl.BlockSpec((1, H), lambda b: (0, 0)),
            pl.BlockSpec((H, H), lambda b: (0, 0)),
            pl.BlockSpec((1, H), lambda b: (0, 0)),
        ],
        out_specs=[
            pl.BlockSpec((1, PADL, N, H), lambda b: (b, 0, 0, 0)),
            pl.BlockSpec((1, NLAYERS, H, N), lambda b: (b, 0, 0, 0)),
        ],
        out_shape=[
            jax.ShapeDtypeStruct((B, PADL, N, H), jnp.float32),
            jax.ShapeDtypeStruct((B, NLAYERS, H, N), jnp.float32),
        ],
    )(x, w1, b1, w2, b2)


HTILE = 64


def _assemble_body(diag_ref, map_ref, mask_ref):
    f32 = jnp.float32
    band = diag_ref[0]
    v = []
    for d in range(NLAYERS):
        t = band[d]
        if d:
            t = jnp.concatenate([t[:, N - d:], t[:, : N - d]], axis=1)
        v.append(t)
    cmod = lax.broadcasted_iota(jnp.int32, (1, N), 1) % NLAYERS
    wlist = []
    for u in range(NLAYERS):
        acc = jnp.where(cmod == 0, v[(-u) % NLAYERS], 0.0)
        for k in range(1, NLAYERS):
            acc = acc + jnp.where(cmod == k, v[(k - u) % NLAYERS], 0.0)
        wlist.append(acc)
    wstack = jnp.stack(wlist, axis=1)
    x = jnp.broadcast_to(
        wstack[:, None], (HTILE, N // NLAYERS, NLAYERS, N)
    ).reshape(HTILE, N, N)
    row2 = lax.broadcasted_iota(jnp.int32, (N, N), 0)
    col2 = lax.broadcasted_iota(jnp.int32, (N, N), 1)
    diff = col2 - row2
    maskf = ((diff >= 0) & (diff < NLAYERS)).astype(f32)
    map_ref[0] = x * maskf[None]
    mask_ref[0, 0] = maskf


def _assemble(band_diag):
    return pl.pallas_call(
        _assemble_body,
        grid=(B, H // HTILE),
        in_specs=[
            pl.BlockSpec((1, NLAYERS, HTILE, N), lambda b, h: (b, 0, h, 0)),
        ],
        out_specs=[
            pl.BlockSpec((1, HTILE, N, N), lambda b, h: (b, h, 0, 0)),
            pl.BlockSpec((1, 1, N, N), lambda b, h: (b, 0, 0, 0)),
        ],
        out_shape=[
            jax.ShapeDtypeStruct((B, H, N, N), jnp.float32),
            jax.ShapeDtypeStruct((B, 1, N, N), jnp.float32),
        ],
    )(band_diag)


_ROWS_PER_B = PADL * N
_ZERO_ROW = NLAYERS * N
_NWORKERS = 32
_PER_W = (B * NPROPS) // _NWORKERS
_CHUNK = 64
_NCH = _PER_W // _CHUNK
_NBUF = 3


def _gather_body(table_hbm, s_hbm, e_hbm, out_hbm,
                 s_v, e_v, idx_v, rows0, rows1, rows2,
                 gs0, gs1, gs2, ws0, ws1, ws2):
    cid = lax.axis_index("c")
    sid = lax.axis_index("s")
    wid = sid * 2 + cid
    b = wid // (NPROPS // _PER_W)
    p0 = (wid % (NPROPS // _PER_W)) * _PER_W
    pltpu.sync_copy(s_hbm.at[pl.ds(p0, _PER_W)], s_v)
    pltpu.sync_copy(e_hbm.at[pl.ds(p0, _PER_W)], e_v)
    base = b * _ROWS_PER_B
    for i in range(_PER_W // 16):
        sv = s_v[pl.ds(i * 16, 16)]
        ev = e_v[pl.ds(i * 16, 16)]
        e1 = ev - 1
        e1 = jnp.where(e1 < 0, e1 + N, e1)
        d = e1 - sv
        valid = (d >= 0) & (d < NLAYERS)
        rows = jnp.where(valid, base + d * N + sv, base + _ZERO_ROW)
        idx_v[i // (_CHUNK // 16), pl.ds((i % (_CHUNK // 16)) * 16, 16)] = rows
    out0 = wid * _PER_W
    bufs = (rows0, rows1, rows2)
    gsems = (gs0, gs1, gs2)
    wsems = (ws0, ws1, ws2)

    def start_gather(t):
        m = t % _NBUF
        return pltpu.async_copy(
            table_hbm.at[idx_v.at[t]],
            bufs[m], gsems[m])

    h_g = [None] * _NCH
    h_w = [None] * _NCH
    for t in range(min(_NBUF - 1, _NCH)):
        h_g[t] = start_gather(t)
    for j in range(_NCH):
        h_g[j].wait()
        h_w[j] = pltpu.async_copy(
            bufs[j % _NBUF], out_hbm.at[pl.ds(out0 + j * _CHUNK, _CHUNK)],
            wsems[j % _NBUF])
        t = j + _NBUF - 1
        if t < _NCH:
            if j >= 1:
                h_w[j - 1].wait()
            h_g[t] = start_gather(t)
    for j in range(max(0, _NCH - _NBUF), _NCH):
        h_w[j].wait()


@functools.cache
def _make_gather():
    return pl.kernel(
        _gather_body,
        mesh=plsc.VectorSubcoreMesh(core_axis_name="c", subcore_axis_name="s"),
        out_type=jax.ShapeDtypeStruct((B * NPROPS, H), jnp.float32),
        scratch_types=[
            pltpu.VMEM((_PER_W,), jnp.int32),
            pltpu.VMEM((_PER_W,), jnp.int32),
            pltpu.VMEM((_NCH, _CHUNK), jnp.int32),
            pltpu.VMEM((_CHUNK, H), jnp.float32),
            pltpu.VMEM((_CHUNK, H), jnp.float32),
            pltpu.VMEM((_CHUNK, H), jnp.float32),
            pltpu.SemaphoreType.DMA,
            pltpu.SemaphoreType.DMA,
            pltpu.SemaphoreType.DMA,
            pltpu.SemaphoreType.DMA,
            pltpu.SemaphoreType.DMA,
            pltpu.SemaphoreType.DMA,
        ],
    )


def kernel(x, props, textual_input, textual_mask, W1, b1, W2, b2):
    del textual_input, textual_mask
    band_pos, band_diag = _chain(
        x, W1, b1.reshape(1, H), W2, b2.reshape(1, H)
    )
    ori_map_h, ori_map_mask = _assemble(band_diag)
    table = band_pos.reshape(B * _ROWS_PER_B, H)
    s_arr = props[:, 0].astype(jnp.int32)
    e_arr = props[:, 1].astype(jnp.int32)
    props_flat = _make_gather()(table, s_arr, e_arr)
    props_h = props_flat.reshape(B, NPROPS, H)
    return props_h, ori_map_h, ori_map_mask
